# baseline (speedup 1.0000x reference)
import functools

import jax
import jax.numpy as jnp
from jax import lax
from jax.experimental import pallas as pl
from jax.experimental.pallas import tpu as pltpu
from jax.experimental.pallas import tpu_sc as plsc

B = 16384
K = 32
L = 16
TW = 128

_info = plsc.get_sparse_core_info()
NC, NS = _info.num_cores, _info.num_subcores
NW = NC * NS
BPW = B // NW
WAVE = 4
N_GROUPS = BPW // L
N_CHUNKS = BPW // L


def _gmf_body(uid_hbm, iid_hbm, user_t, item_t, wb_hbm, out_hbm,
              uidx, iidx, ublk, iblk, grows, girows, outv, wbv,
              sem_idx, sem_a, sem_b, sem_c):
    wid = lax.axis_index("s") * NC + lax.axis_index("c")
    base = wid * BPW

    cp_u = pltpu.make_async_copy(uid_hbm.at[pl.ds(base, BPW)], uidx, sem_idx)
    cp_i = pltpu.make_async_copy(iid_hbm.at[pl.ds(base, BPW)], iidx, sem_idx)
    cp_u.start()
    cp_i.start()
    pltpu.sync_copy(wb_hbm, wbv)
    cp_u.wait()
    cp_i.wait()

    iota = lax.iota(jnp.int32, L)
    lo = iota % jnp.int32(L)
    hi = iota + jnp.int32(L)

    def _fetch_wave(uvec, ivec, w, ring, sem):
        for l in range(WAVE):
            ru = uvec[w * WAVE + l]
            ri = ivec[w * WAVE + l]
            bu = pl.multiple_of((ru // TW) * TW, TW)
            bi = pl.multiple_of((ri // TW) * TW, TW)
            pltpu.make_async_copy(
                user_t.at[:, pl.ds(bu, TW)], ublk.at[ring, l], sem).start()
            pltpu.make_async_copy(
                item_t.at[:, pl.ds(bi, TW)], iblk.at[ring, l], sem).start()

    def _drain_wave(ring, sem):
        dummy = user_t.at[pl.ds(0, K), pl.ds(0, TW)]
        for l in range(WAVE):
            pltpu.make_async_copy(dummy, ublk.at[ring, l], sem).wait()
            pltpu.make_async_copy(dummy, iblk.at[ring, l], sem).wait()

    def _extract_wave(uvec, ivec, w, ring):
        for l in range(WAVE):
            p = w * WAVE + l
            cu = jnp.full((L,), uvec[w * WAVE + l] % TW, jnp.int32)
            ci = jnp.full((L,), ivec[w * WAVE + l] % TW, jnp.int32)
            grows[p, pl.ds(0, L)] = plsc.load_gather(ublk.at[ring, l], [lo, cu])
            grows[p, pl.ds(L, L)] = plsc.load_gather(ublk.at[ring, l], [hi, cu])
            girows[p, pl.ds(0, L)] = plsc.load_gather(iblk.at[ring, l], [lo, ci])
            girows[p, pl.ds(L, L)] = plsc.load_gather(iblk.at[ring, l], [hi, ci])

    n_waves = L // WAVE

    w0 = wbv[0, pl.ds(0, L)]
    w1 = wbv[0, pl.ds(L, L)]
    bvec = wbv[0, pl.ds(K, L)]
    wk = [jnp.full((L,), (w0 if k < L else w1)[k % L], jnp.float32)
          for k in range(K)]
    bias = jnp.full((L,), bvec[0], jnp.float32)

    sems = (sem_a, sem_b, sem_c)

    def group_body(g, carry):
        uvec = uidx[pl.ds(g * L, L)]
        ivec = iidx[pl.ds(g * L, L)]
        _fetch_wave(uvec, ivec, 0, 0, sems[0])
        _fetch_wave(uvec, ivec, 1, 1, sems[1])
        for w in range(n_waves):
            ring = w % 3
            if w + 2 < n_waves:
                nring = (w + 2) % 3
                _fetch_wave(uvec, ivec, w + 2, nring, sems[nring])
            _drain_wave(ring, sems[ring])
            _extract_wave(uvec, ivec, w, ring)
        acc0 = bias
        acc1 = jnp.zeros((L,), jnp.float32)
        for k in range(K):
            kidx = jnp.full((L,), k, jnp.int32)
            term = (plsc.load_gather(grows, [iota, kidx])
                    * plsc.load_gather(girows, [iota, kidx]) * wk[k])
            if k % 2 == 0:
                acc0 = acc0 + term
            else:
                acc1 = acc1 + term
        acc = acc0 + acc1
        outv[pl.ds(g * L, L)] = 1.0 / (1.0 + jnp.exp(-acc))
        return carry

    lax.fori_loop(0, N_GROUPS, group_body, 0)
    pltpu.sync_copy(outv, out_hbm.at[pl.ds(base, BPW)])


@jax.jit
def kernel(uid, iid, user_mat, item_mat, affine_w, affine_b):
    mesh = plsc.VectorSubcoreMesh(core_axis_name="c", subcore_axis_name="s")
    f = pl.kernel(
        _gmf_body,
        out_type=jax.ShapeDtypeStruct((B,), jnp.float32),
        mesh=mesh,
        compiler_params=pltpu.CompilerParams(needs_layout_passes=False),
        scratch_types=[
            pltpu.VMEM((BPW,), jnp.int32),
            pltpu.VMEM((BPW,), jnp.int32),
            pltpu.VMEM((3, WAVE, K, TW), jnp.float32),
            pltpu.VMEM((3, WAVE, K, TW), jnp.float32),
            pltpu.VMEM((L, K), jnp.float32),
            pltpu.VMEM((L, K), jnp.float32),
            pltpu.VMEM((BPW,), jnp.float32),
            pltpu.VMEM((1, 128), jnp.float32),
            pltpu.SemaphoreType.DMA,
            pltpu.SemaphoreType.DMA,
            pltpu.SemaphoreType.DMA,
            pltpu.SemaphoreType.DMA,
        ],
    )
    wb = jnp.zeros((128,), jnp.float32)
    wb = wb.at[:K].set(affine_w[0]).at[K].set(affine_b[0]).reshape(1, 128)
    return f(uid, iid, user_mat.T, item_mat.T, wb)


# speedup vs baseline: 1.0038x; 1.0038x over previous
import functools

import jax
import jax.numpy as jnp
from jax import lax
from jax.experimental import pallas as pl
from jax.experimental.pallas import tpu as pltpu
from jax.experimental.pallas import tpu_sc as plsc

B = 16384
K = 32
L = 16
TW = 128

_info = plsc.get_sparse_core_info()
NC, NS = _info.num_cores, _info.num_subcores
NW = NC * NS
BPW = B // NW
WAVE = 4
N_GROUPS = BPW // L
N_CHUNKS = BPW // L


def _gmf_body(uid_hbm, iid_hbm, user_t, item_t, wb_hbm, out_hbm,
              uidx, iidx, ublk, iblk, grows, girows, outv, wbv,
              sem_idx, sem_a, sem_b, sem_c):
    wid = lax.axis_index("s") * NC + lax.axis_index("c")
    base = wid * BPW

    cp_u = pltpu.make_async_copy(uid_hbm.at[pl.ds(base, BPW)], uidx, sem_idx)
    cp_i = pltpu.make_async_copy(iid_hbm.at[pl.ds(base, BPW)], iidx, sem_idx)
    cp_u.start()
    cp_i.start()
    pltpu.sync_copy(wb_hbm, wbv)
    cp_u.wait()
    cp_i.wait()

    iota = lax.iota(jnp.int32, L)
    lo = iota % jnp.int32(L)
    hi = iota + jnp.int32(L)

    def _fetch_wave(uvec, ivec, w, ring, sem):
        for l in range(WAVE):
            ru = uvec[w * WAVE + l]
            ri = ivec[w * WAVE + l]
            bu = pl.multiple_of((ru // TW) * TW, TW)
            bi = pl.multiple_of((ri // TW) * TW, TW)
            for i in range(K // 8):
                pltpu.make_async_copy(
                    user_t.at[i, :, pl.ds(bu, TW)],
                    ublk.at[ring, l, i], sem).start()
                pltpu.make_async_copy(
                    item_t.at[i, :, pl.ds(bi, TW)],
                    iblk.at[ring, l, i], sem).start()

    def _drain_wave(ring, sem):
        dummy = user_t.at[:, :, pl.ds(0, TW)]
        for l in range(WAVE):
            pltpu.make_async_copy(dummy, ublk.at[ring, l], sem).wait()
            pltpu.make_async_copy(dummy, iblk.at[ring, l], sem).wait()

    def _extract_wave(uvec, ivec, w, ring):
        for l in range(WAVE):
            p = w * WAVE + l
            cu = jnp.full((L,), uvec[w * WAVE + l] % TW, jnp.int32)
            ci = jnp.full((L,), ivec[w * WAVE + l] % TW, jnp.int32)
            ub = ublk.at[ring, l]
            ib = iblk.at[ring, l]
            grows[p, pl.ds(0, L)] = plsc.load_gather(
                ub, [lo // 8, lo % 8, cu])
            grows[p, pl.ds(L, L)] = plsc.load_gather(
                ub, [hi // 8, hi % 8, cu])
            girows[p, pl.ds(0, L)] = plsc.load_gather(
                ib, [lo // 8, lo % 8, ci])
            girows[p, pl.ds(L, L)] = plsc.load_gather(
                ib, [hi // 8, hi % 8, ci])

    n_waves = L // WAVE

    w0 = wbv[0, pl.ds(0, L)]
    w1 = wbv[0, pl.ds(L, L)]
    bvec = wbv[0, pl.ds(K, L)]
    wk = [jnp.full((L,), (w0 if k < L else w1)[k % L], jnp.float32)
          for k in range(K)]
    bias = jnp.full((L,), bvec[0], jnp.float32)

    sems = (sem_a, sem_b, sem_c)

    def group_body(g, carry):
        uvec = uidx[pl.ds(g * L, L)]
        ivec = iidx[pl.ds(g * L, L)]
        _fetch_wave(uvec, ivec, 0, 0, sems[0])
        _fetch_wave(uvec, ivec, 1, 1, sems[1])
        for w in range(n_waves):
            ring = w % 3
            if w + 2 < n_waves:
                nring = (w + 2) % 3
                _fetch_wave(uvec, ivec, w + 2, nring, sems[nring])
            _drain_wave(ring, sems[ring])
            _extract_wave(uvec, ivec, w, ring)
        acc0 = bias
        acc1 = jnp.zeros((L,), jnp.float32)
        for k in range(K):
            kidx = jnp.full((L,), k, jnp.int32)
            term = (plsc.load_gather(grows, [iota, kidx])
                    * plsc.load_gather(girows, [iota, kidx]) * wk[k])
            if k % 2 == 0:
                acc0 = acc0 + term
            else:
                acc1 = acc1 + term
        acc = acc0 + acc1
        outv[pl.ds(g * L, L)] = 1.0 / (1.0 + jnp.exp(-acc))
        return carry

    lax.fori_loop(0, N_GROUPS, group_body, 0)
    pltpu.sync_copy(outv, out_hbm.at[pl.ds(base, BPW)])


@jax.jit
def kernel(uid, iid, user_mat, item_mat, affine_w, affine_b):
    mesh = plsc.VectorSubcoreMesh(core_axis_name="c", subcore_axis_name="s")
    f = pl.kernel(
        _gmf_body,
        out_type=jax.ShapeDtypeStruct((B,), jnp.float32),
        mesh=mesh,
        compiler_params=pltpu.CompilerParams(needs_layout_passes=False),
        scratch_types=[
            pltpu.VMEM((BPW,), jnp.int32),
            pltpu.VMEM((BPW,), jnp.int32),
            pltpu.VMEM((3, WAVE, K // 8, 8, TW), jnp.float32),
            pltpu.VMEM((3, WAVE, K // 8, 8, TW), jnp.float32),
            pltpu.VMEM((L, K), jnp.float32),
            pltpu.VMEM((L, K), jnp.float32),
            pltpu.VMEM((BPW,), jnp.float32),
            pltpu.VMEM((1, 128), jnp.float32),
            pltpu.SemaphoreType.DMA,
            pltpu.SemaphoreType.DMA,
            pltpu.SemaphoreType.DMA,
            pltpu.SemaphoreType.DMA,
        ],
    )
    wb = jnp.zeros((128,), jnp.float32)
    wb = wb.at[:K].set(affine_w[0]).at[K].set(affine_b[0]).reshape(1, 128)
    ut = user_mat.T.reshape(K // 8, 8, user_mat.shape[0])
    it = item_mat.T.reshape(K // 8, 8, item_mat.shape[0])
    return f(uid, iid, ut, it, wb)


# speedup vs baseline: 1.0150x; 1.0112x over previous
import jax
import jax.numpy as jnp
from jax import lax
from jax.experimental import pallas as pl
from jax.experimental.pallas import tpu as pltpu
from jax.experimental.pallas import tpu_sc as plsc

B = 16384
K = 32
L = 16
TW = 128

_info = plsc.get_sparse_core_info()
NC, NS = _info.num_cores, _info.num_subcores
NW = NC * NS
BPW = B // NW
WAVE = 4
N_GROUPS = BPW // L
N_CHUNKS = BPW // L


def _gmf_body(uid_hbm, iid_hbm, user_t, item_t, wb_hbm, out_hbm,
              uidx, iidx, ublk, iblk, grows, girows, outv, wbv,
              sem_idx, sem_a, sem_b):
    wid = lax.axis_index("s") * NC + lax.axis_index("c")
    base = wid * BPW

    cp_u = pltpu.make_async_copy(uid_hbm.at[pl.ds(base, BPW)], uidx, sem_idx)
    cp_i = pltpu.make_async_copy(iid_hbm.at[pl.ds(base, BPW)], iidx, sem_idx)
    cp_u.start()
    cp_i.start()
    pltpu.sync_copy(wb_hbm, wbv)
    cp_u.wait()
    cp_i.wait()

    iota = lax.iota(jnp.int32, L)
    lo = iota
    hi = iota + jnp.int32(L)

    def _fetch_wave(uvec, ivec, w, ring, sem):
        for l in range(WAVE):
            ru = uvec[w * WAVE + l]
            ri = ivec[w * WAVE + l]
            bu = pl.multiple_of((ru // TW) * TW, TW)
            bi = pl.multiple_of((ri // TW) * TW, TW)
            pltpu.make_async_copy(
                user_t.at[:, pl.ds(bu, TW)], ublk.at[ring, l], sem).start()
            pltpu.make_async_copy(
                item_t.at[:, pl.ds(bi, TW)], iblk.at[ring, l], sem).start()

    def _drain_wave(ring, sem):
        dummy = user_t.at[pl.ds(0, K), pl.ds(0, TW)]
        for l in range(WAVE):
            pltpu.make_async_copy(dummy, ublk.at[ring, l], sem).wait()
            pltpu.make_async_copy(dummy, iblk.at[ring, l], sem).wait()

    def _extract_wave(uvec, ivec, w, ring):
        for l in range(WAVE):
            p = w * WAVE + l
            cu = jnp.full((L,), uvec[w * WAVE + l] % TW, jnp.int32)
            ci = jnp.full((L,), ivec[w * WAVE + l] % TW, jnp.int32)
            grows[p, pl.ds(0, L)] = plsc.load_gather(ublk.at[ring, l], [lo, cu])
            grows[p, pl.ds(L, L)] = plsc.load_gather(ublk.at[ring, l], [hi, cu])
            girows[p, pl.ds(0, L)] = plsc.load_gather(iblk.at[ring, l], [lo, ci])
            girows[p, pl.ds(L, L)] = plsc.load_gather(iblk.at[ring, l], [hi, ci])

    n_waves = L // WAVE

    w0 = wbv[0, pl.ds(0, L)]
    w1 = wbv[0, pl.ds(L, L)]
    bvec = wbv[0, pl.ds(K, L)]
    wk = [jnp.full((L,), (w0 if k < L else w1)[k % L], jnp.float32)
          for k in range(K)]
    bias = jnp.full((L,), bvec[0], jnp.float32)

    def group_body(g, carry):
        uvec = uidx[pl.ds(g * L, L)]
        ivec = iidx[pl.ds(g * L, L)]
        _fetch_wave(uvec, ivec, 0, 0, sem_a)
        for w in range(n_waves):
            ring, sem = w % 2, (sem_a if w % 2 == 0 else sem_b)
            if w + 1 < n_waves:
                nring = (w + 1) % 2
                nsem = sem_a if (w + 1) % 2 == 0 else sem_b
                _fetch_wave(uvec, ivec, w + 1, nring, nsem)
            _drain_wave(ring, sem)
            _extract_wave(uvec, ivec, w, ring)
        acc0 = bias
        acc1 = jnp.zeros((L,), jnp.float32)
        for k in range(K):
            kidx = jnp.full((L,), k, jnp.int32)
            term = (plsc.load_gather(grows, [iota, kidx])
                    * plsc.load_gather(girows, [iota, kidx]) * wk[k])
            if k % 2 == 0:
                acc0 = acc0 + term
            else:
                acc1 = acc1 + term
        acc = acc0 + acc1
        outv[pl.ds(g * L, L)] = 1.0 / (1.0 + jnp.exp(-acc))
        return carry

    lax.fori_loop(0, N_GROUPS, group_body, 0)
    pltpu.sync_copy(outv, out_hbm.at[pl.ds(base, BPW)])


@jax.jit
def kernel(uid, iid, user_mat, item_mat, affine_w, affine_b):
    mesh = plsc.VectorSubcoreMesh(core_axis_name="c", subcore_axis_name="s")
    f = pl.kernel(
        _gmf_body,
        out_type=jax.ShapeDtypeStruct((B,), jnp.float32),
        mesh=mesh,
        compiler_params=pltpu.CompilerParams(needs_layout_passes=False),
        scratch_types=[
            pltpu.VMEM((BPW,), jnp.int32),
            pltpu.VMEM((BPW,), jnp.int32),
            pltpu.VMEM((2, WAVE, K, TW), jnp.float32),
            pltpu.VMEM((2, WAVE, K, TW), jnp.float32),
            pltpu.VMEM((L, K), jnp.float32),
            pltpu.VMEM((L, K), jnp.float32),
            pltpu.VMEM((BPW,), jnp.float32),
            pltpu.VMEM((1, 128), jnp.float32),
            pltpu.SemaphoreType.DMA,
            pltpu.SemaphoreType.DMA,
            pltpu.SemaphoreType.DMA,
        ],
    )
    wb = jnp.zeros((128,), jnp.float32)
    wb = wb.at[:K].set(affine_w[0]).at[K].set(affine_b[0]).reshape(1, 128)
    return f(uid, iid, user_mat.T, item_mat.T, wb)
